# baseline (speedup 1.0000x reference)
import functools

import jax
import jax.numpy as jnp
from jax import lax
from jax.experimental import pallas as pl
from jax.experimental.pallas import tpu as pltpu
from jax.experimental.pallas import tpu_sc as plsc

NUM_CORES = 2
NUM_SUBCORES = 16
LANES = 16
NUM_WORKERS = NUM_CORES * NUM_SUBCORES

VOCAB = 100000
EMBED_DIM = 64
NUM_TOKENS = 2
B, T = 4096, 50

VA = (VOCAB // 128) * 128
NBLK = VA // 128
TAIL = VOCAB - VA
BLK_PER_SUB = -(-NBLK // NUM_SUBCORES)
BPW = B // NUM_WORKERS
QR = EMBED_DIM // LANES


def _make_kernel():
    mesh = plsc.VectorSubcoreMesh(
        core_axis_name="c", subcore_axis_name="s",
        num_cores=NUM_CORES, num_subcores=NUM_SUBCORES)

    @functools.partial(
        pl.kernel,
        out_type=(
            jax.ShapeDtypeStruct((T, NUM_TOKENS, EMBED_DIM, B), jnp.float32),
            jax.ShapeDtypeStruct((NUM_CORES, VOCAB, 128), jnp.float32),
        ),
        mesh=mesh,
        scratch_types=[
            pltpu.VMEM((2, EMBED_DIM, 128), jnp.float32),
            pltpu.VMEM((2, 128, 128), jnp.float32),
            pltpu.VMEM((T, BPW), jnp.int32),
            pltpu.VMEM((2, BPW, 128), jnp.float32),
            pltpu.VMEM((2, NUM_TOKENS, EMBED_DIM, BPW), jnp.float32),
            pltpu.VMEM((NUM_TOKENS, EMBED_DIM), jnp.float32),
            pltpu.SemaphoreType.DMA,
            pltpu.SemaphoreType.DMA,
            pltpu.SemaphoreType.DMA,
            pltpu.SemaphoreType.DMA,
        ],
        compiler_params=pltpu.CompilerParams(needs_layout_passes=False),
    )
    def action_encode(actions_hbm, table_hbm, lt_hbm, tail_hbm,
                      out_hbm, scr_hbm,
                      slab_v, tr_v, idx_v, rows_v, out_v, lt_v,
                      sem_g0, sem_g1, sem_w0, sem_w1):
        cid = lax.axis_index("c")
        sid = lax.axis_index("s")
        sem_g = [sem_g0, sem_g1]
        sem_w = [sem_w0, sem_w1]

        iota = lax.iota(jnp.int32, LANES)
        rvec = [iota + (q * LANES) for q in range(8)]

        my_scr = scr_hbm.at[cid]

        def scr_slice(j):
            return my_scr.at[pl.ds(j * 128, 128)]

        def p1_read(j, b):
            return pltpu.make_async_copy(
                table_hbm.at[:, pl.ds(j * 128, 128)], slab_v.at[b], sem_g[b])

        def p1_write(j, b):
            return pltpu.make_async_copy(tr_v.at[b], scr_slice(j), sem_w[b])

        def transpose_block(b):
            slab_b = slab_v.at[b]
            tr_b = tr_v.at[b]

            def c_body(c, carry):
                bc = jnp.full((LANES,), c, jnp.int32)
                for q in range(QR):
                    g = plsc.load_gather(slab_b, [rvec[q], bc])
                    tr_b[c, pl.ds(q * LANES, LANES)] = g
                    tr_b[c, pl.ds(EMBED_DIM + q * LANES, LANES)] = g
                return carry

            lax.fori_loop(0, 128, c_body, 0, unroll=2)

        def p1_step(jj):
            b = jj & 1
            j = jj * NUM_SUBCORES + sid
            if jj + 1 < BLK_PER_SUB:
                j2 = j + NUM_SUBCORES
                if jj + 1 == BLK_PER_SUB - 1:
                    @pl.when(j2 < NBLK)
                    def _pf():
                        p1_read(j2, 1 - b).start()
                else:
                    p1_read(j2, 1 - b).start()

            def body():
                p1_read(j, b).wait()
                if jj >= 2:
                    p1_write(j - 2 * NUM_SUBCORES, b).wait()
                transpose_block(b)
                p1_write(j, b).start()

            if jj == BLK_PER_SUB - 1:
                @pl.when(j < NBLK)
                def _guarded():
                    body()
            else:
                body()

        p1_read(sid, 0).start()
        for jj in range(BLK_PER_SUB):
            p1_step(jj)

        jlast0 = (BLK_PER_SUB - 1) * NUM_SUBCORES + sid

        @pl.when(jlast0 < NBLK)
        def _w0a():
            p1_write(jlast0, 0).wait()

        @pl.when(jlast0 >= NBLK)
        def _w0b():
            p1_write(jlast0 - 2 * NUM_SUBCORES, 0).wait()

        p1_write((BLK_PER_SUB - 2) * NUM_SUBCORES + sid, 1).wait()

        @pl.when(sid == 0)
        def _tail():
            tv = tr_v.at[0].at[pl.ds(0, TAIL), :]
            pltpu.sync_copy(tail_hbm, tv)
            pltpu.sync_copy(tv, my_scr.at[pl.ds(VA, TAIL)])

        plsc.subcore_barrier()

        w = sid * NUM_CORES + cid
        b0 = w * BPW

        pltpu.sync_copy(lt_hbm, lt_v)
        pltpu.sync_copy(actions_hbm.at[:, pl.ds(b0, BPW)], idx_v)
        lt0 = [lt_v[0, pl.ds(q * LANES, LANES)] for q in range(QR)]
        lt1 = [lt_v[1, pl.ds(q * LANES, LANES)] for q in range(QR)]
        zvec = jnp.zeros((LANES,), jnp.int32)
        onevec = jnp.full((LANES,), 1, jnp.int32)

        def gather(t, b):
            return pltpu.make_async_copy(
                my_scr.at[idx_v.at[t]], rows_v.at[b], sem_g[b])

        def wb(t, b):
            return pltpu.make_async_copy(
                out_v.at[b], out_hbm.at[t, :, :, pl.ds(b0, BPW)], sem_w[b])

        def compute(b):
            rows_b = rows_v.at[b]
            out_b = out_v.at[b]

            def j_body(j, carry):
                sj = jnp.full((LANES,), j, jnp.int32)
                for q in range(QR):
                    r = rows_b[j, pl.ds(q * LANES, LANES)]
                    plsc.store_scatter(out_b, [zvec, rvec[q], sj], r + lt0[q])
                    plsc.store_scatter(out_b, [onevec, rvec[q], sj], r + lt1[q])
                return carry

            lax.fori_loop(0, BPW, j_body, 0, unroll=2)

        def step(t, b, do_wait_wb, do_gather_ahead):
            if do_gather_ahead:
                gather(t + 1, 1 - b).start()
            gather(t, b).wait()
            if do_wait_wb:
                wb(t - 2, b).wait()
            compute(b)
            wb(t, b).start()

        gather(0, 0).start()
        step(0, 0, do_wait_wb=False, do_gather_ahead=True)
        step(1, 1, do_wait_wb=False, do_gather_ahead=True)

        @pl.loop(2, T - 2, step=2)
        def _steady(t0):
            step(t0, 0, do_wait_wb=True, do_gather_ahead=True)
            step(t0 + 1, 1, do_wait_wb=True, do_gather_ahead=True)

        step(T - 2, 0, do_wait_wb=True, do_gather_ahead=True)
        step(T - 1, 1, do_wait_wb=True, do_gather_ahead=False)
        wb(T - 2, 0).wait()
        wb(T - 1, 1).wait()

    return action_encode


def kernel(actions, action_embed, learned_token):
    actions_t = actions.T.astype(jnp.int32)
    table_t = action_embed.T
    lt = learned_token.reshape(NUM_TOKENS, EMBED_DIM)
    tail = jnp.concatenate([action_embed[VA:], action_embed[VA:]], axis=1)
    out, _ = _make_kernel()(actions_t, table_t, lt, tail)
    return jnp.transpose(out, (3, 0, 1, 2))


# speedup vs baseline: 1.4190x; 1.4190x over previous
import functools

import jax
import jax.numpy as jnp
from jax import lax
from jax.experimental import pallas as pl
from jax.experimental.pallas import tpu as pltpu
from jax.experimental.pallas import tpu_sc as plsc

NUM_CORES = 2
NUM_SUBCORES = 16
LANES = 16
NUM_WORKERS = NUM_CORES * NUM_SUBCORES

VOCAB = 100000
EMBED_DIM = 64
NUM_TOKENS = 2
B, T = 4096, 50

VA = (VOCAB // 128) * 128
NBLK = VA // 128
TAIL = VOCAB - VA
BLK_PER_SUB = -(-NBLK // NUM_SUBCORES)
BPW = B // NUM_WORKERS
QR = EMBED_DIM // LANES


def _make_kernel():
    mesh = plsc.VectorSubcoreMesh(
        core_axis_name="c", subcore_axis_name="s",
        num_cores=NUM_CORES, num_subcores=NUM_SUBCORES)

    @functools.partial(
        pl.kernel,
        out_type=(
            jax.ShapeDtypeStruct((T, NUM_TOKENS, EMBED_DIM, B), jnp.float32),
            jax.ShapeDtypeStruct((NUM_CORES, VOCAB, 128), jnp.float32),
        ),
        mesh=mesh,
        scratch_types=[
            pltpu.VMEM((2, EMBED_DIM, 128), jnp.float32),
            pltpu.VMEM((2, 128, 128), jnp.float32),
            pltpu.VMEM((T, BPW), jnp.int32),
            pltpu.VMEM((2, BPW, 128), jnp.float32),
            pltpu.VMEM((2, NUM_TOKENS, EMBED_DIM, BPW), jnp.float32),
            pltpu.VMEM((NUM_TOKENS, EMBED_DIM), jnp.float32),
            pltpu.SemaphoreType.DMA,
            pltpu.SemaphoreType.DMA,
            pltpu.SemaphoreType.DMA,
            pltpu.SemaphoreType.DMA,
        ],
        compiler_params=pltpu.CompilerParams(needs_layout_passes=False),
    )
    def action_encode(actions_hbm, table_hbm, lt_hbm, tail_hbm,
                      out_hbm, scr_hbm,
                      slab_v, tr_v, idx_v, rows_v, out_v, lt_v,
                      sem_g0, sem_g1, sem_w0, sem_w1):
        cid = lax.axis_index("c")
        sid = lax.axis_index("s")
        sem_g = [sem_g0, sem_g1]
        sem_w = [sem_w0, sem_w1]

        iota = lax.iota(jnp.int32, LANES)
        rvec = [iota + (q * LANES) for q in range(8)]

        my_scr = scr_hbm.at[cid]

        def scr_slice(j):
            return my_scr.at[pl.ds(j * 128, 128)]

        def p1_read(j, b):
            return pltpu.make_async_copy(
                table_hbm.at[:, pl.ds(j * 128, 128)], slab_v.at[b], sem_g[b])

        def p1_write(j, b):
            return pltpu.make_async_copy(tr_v.at[b], scr_slice(j), sem_w[b])

        def transpose_block(b):
            slab_b = slab_v.at[b]
            tr_b = tr_v.at[b]

            @plsc.parallel_loop(0, 128, unroll=4)
            def _c_body(c):
                bc = jnp.full((LANES,), c, jnp.int32)
                for q in range(QR):
                    g = plsc.load_gather(slab_b, [rvec[q], bc])
                    tr_b[c, pl.ds(q * LANES, LANES)] = g
                    tr_b[c, pl.ds(EMBED_DIM + q * LANES, LANES)] = g

        def p1_step(jj):
            b = jj & 1
            j = jj * NUM_SUBCORES + sid
            if jj + 1 < BLK_PER_SUB:
                j2 = j + NUM_SUBCORES
                if jj + 1 == BLK_PER_SUB - 1:
                    @pl.when(j2 < NBLK)
                    def _pf():
                        p1_read(j2, 1 - b).start()
                else:
                    p1_read(j2, 1 - b).start()

            def front():
                p1_read(j, b).wait()
                if jj >= 2:
                    p1_write(j - 2 * NUM_SUBCORES, b).wait()
                transpose_block(b)

            def back():
                p1_write(j, b).start()

            if jj == BLK_PER_SUB - 1:
                @pl.when(j < NBLK)
                def _gf():
                    front()
                plsc.subcore_barrier()

                @pl.when(j < NBLK)
                def _gb():
                    back()
            else:
                front()
                plsc.subcore_barrier()
                back()

        p1_read(sid, 0).start()
        for jj in range(BLK_PER_SUB):
            p1_step(jj)

        jlast0 = (BLK_PER_SUB - 1) * NUM_SUBCORES + sid

        @pl.when(jlast0 < NBLK)
        def _w0a():
            p1_write(jlast0, 0).wait()

        @pl.when(jlast0 >= NBLK)
        def _w0b():
            p1_write(jlast0 - 2 * NUM_SUBCORES, 0).wait()

        p1_write((BLK_PER_SUB - 2) * NUM_SUBCORES + sid, 1).wait()

        @pl.when(sid == 0)
        def _tail():
            tv = tr_v.at[0].at[pl.ds(0, TAIL), :]
            pltpu.sync_copy(tail_hbm, tv)
            pltpu.sync_copy(tv, my_scr.at[pl.ds(VA, TAIL)])

        plsc.subcore_barrier()

        w = sid * NUM_CORES + cid
        b0 = w * BPW

        pltpu.sync_copy(lt_hbm, lt_v)
        pltpu.sync_copy(actions_hbm.at[:, pl.ds(b0, BPW)], idx_v)
        lt0 = [lt_v[0, pl.ds(q * LANES, LANES)] for q in range(QR)]
        lt1 = [lt_v[1, pl.ds(q * LANES, LANES)] for q in range(QR)]
        zvec = jnp.zeros((LANES,), jnp.int32)
        onevec = jnp.full((LANES,), 1, jnp.int32)

        def gather(t, b):
            return pltpu.make_async_copy(
                my_scr.at[idx_v.at[t]], rows_v.at[b], sem_g[b])

        def wb(t, b):
            return pltpu.make_async_copy(
                out_v.at[b], out_hbm.at[t, :, :, pl.ds(b0, BPW)], sem_w[b])

        def compute(b):
            rows_b = rows_v.at[b]
            out_b = out_v.at[b]

            @plsc.parallel_loop(0, BPW, unroll=4)
            def _j_body(j):
                sj = jnp.full((LANES,), j, jnp.int32)
                for q in range(QR):
                    r = rows_b[j, pl.ds(q * LANES, LANES)]
                    plsc.store_scatter(out_b, [zvec, rvec[q], sj], r + lt0[q])
                    plsc.store_scatter(out_b, [onevec, rvec[q], sj], r + lt1[q])

            plsc.subcore_barrier()

        def step(t, b, do_wait_wb, do_gather_ahead):
            if do_gather_ahead:
                gather(t + 1, 1 - b).start()
            gather(t, b).wait()
            if do_wait_wb:
                wb(t - 2, b).wait()
            compute(b)
            wb(t, b).start()

        gather(0, 0).start()
        step(0, 0, do_wait_wb=False, do_gather_ahead=True)
        step(1, 1, do_wait_wb=False, do_gather_ahead=True)

        @pl.loop(2, T - 2, step=2)
        def _steady(t0):
            step(t0, 0, do_wait_wb=True, do_gather_ahead=True)
            step(t0 + 1, 1, do_wait_wb=True, do_gather_ahead=True)

        step(T - 2, 0, do_wait_wb=True, do_gather_ahead=True)
        step(T - 1, 1, do_wait_wb=True, do_gather_ahead=False)
        wb(T - 2, 0).wait()
        wb(T - 1, 1).wait()

    return action_encode


def kernel(actions, action_embed, learned_token):
    actions_t = actions.T.astype(jnp.int32)
    table_t = action_embed.T
    lt = learned_token.reshape(NUM_TOKENS, EMBED_DIM)
    tail = jnp.concatenate([action_embed[VA:], action_embed[VA:]], axis=1)
    out, _ = _make_kernel()(actions_t, table_t, lt, tail)
    return jnp.transpose(out, (3, 0, 1, 2))


# speedup vs baseline: 2.2957x; 1.6178x over previous
import functools

import jax
import jax.numpy as jnp
from jax import lax
from jax.experimental import pallas as pl
from jax.experimental.pallas import tpu as pltpu
from jax.experimental.pallas import tpu_sc as plsc

NUM_CORES = 2
NUM_SUBCORES = 16
LANES = 16
NUM_WORKERS = NUM_CORES * NUM_SUBCORES

VOCAB = 100000
EMBED_DIM = 64
NUM_TOKENS = 2
B, T = 4096, 50

VA = (VOCAB // 128) * 128
NBLK = VA // 128
TAIL = VOCAB - VA
BLK_PER_SUB = -(-NBLK // NUM_SUBCORES)
BPW = B // NUM_WORKERS
QR = EMBED_DIM // LANES


def _make_kernel():
    mesh = plsc.VectorSubcoreMesh(
        core_axis_name="c", subcore_axis_name="s",
        num_cores=NUM_CORES, num_subcores=NUM_SUBCORES)

    @functools.partial(
        pl.kernel,
        out_type=(
            jax.ShapeDtypeStruct((T, NUM_TOKENS, EMBED_DIM, B), jnp.float32),
            jax.ShapeDtypeStruct((NUM_CORES, VOCAB // 2, 128), jnp.float32),
        ),
        mesh=mesh,
        scratch_types=[
            pltpu.VMEM((2, EMBED_DIM, 128), jnp.float32),
            pltpu.VMEM((2, EMBED_DIM, 128), jnp.float32),
            pltpu.VMEM((T, BPW), jnp.int32),
            pltpu.VMEM((T, BPW), jnp.int32),
            pltpu.VMEM((T, BPW), jnp.int32),
            pltpu.VMEM((2, BPW, 128), jnp.float32),
            pltpu.VMEM((2, NUM_TOKENS, EMBED_DIM, BPW), jnp.float32),
            pltpu.VMEM((NUM_TOKENS, EMBED_DIM), jnp.float32),
            pltpu.SemaphoreType.DMA,
            pltpu.SemaphoreType.DMA,
            pltpu.SemaphoreType.DMA,
            pltpu.SemaphoreType.DMA,
        ],
        compiler_params=pltpu.CompilerParams(needs_layout_passes=False),
    )
    def action_encode(actions_hbm, table_hbm, lt_hbm, tail_hbm,
                      out_hbm, scr_hbm,
                      slab_v, tr_v, idx_v, ihalf_v, offv_v, rows_v, out_v, lt_v,
                      sem_g0, sem_g1, sem_w0, sem_w1):
        cid = lax.axis_index("c")
        sid = lax.axis_index("s")
        sem_g = [sem_g0, sem_g1]
        sem_w = [sem_w0, sem_w1]

        iota = lax.iota(jnp.int32, LANES)
        rvec = [iota + (q * LANES) for q in range(8)]

        my_scr = scr_hbm.at[cid]

        def scr_slice(j):
            return my_scr.at[pl.ds(j * EMBED_DIM, EMBED_DIM)]

        def p1_read(j, b):
            return pltpu.make_async_copy(
                table_hbm.at[:, pl.ds(j * 128, 128)], slab_v.at[b], sem_g[b])

        def p1_write(j, b):
            return pltpu.make_async_copy(tr_v.at[b], scr_slice(j), sem_w[b])

        def transpose_block(b):
            slab_b = slab_v.at[b]
            tr_b = tr_v.at[b]

            @plsc.parallel_loop(0, EMBED_DIM, unroll=4)
            def _c_body(c):
                bc0 = jnp.full((LANES,), 2 * c, jnp.int32)
                bc1 = bc0 + 1
                for q in range(2 * QR):
                    g = plsc.load_gather(
                        slab_b, [rvec[q % QR], bc0 if q < QR else bc1])
                    tr_b[c, pl.ds(q * LANES, LANES)] = g

        def p1_step(jj, b, is_last):
            j = jj * NUM_SUBCORES + sid
            j2 = j + NUM_SUBCORES

            @pl.when(j2 < NBLK)
            def _pf():
                p1_read(j2, 1 - b).start()

            def front():
                p1_read(j, b).wait()
                if isinstance(jj, int):
                    if jj >= 2:
                        p1_write(j - 2 * NUM_SUBCORES, b).wait()
                else:
                    @pl.when(jj >= 2)
                    def _ww():
                        p1_write(j - 2 * NUM_SUBCORES, b).wait()

                transpose_block(b)

            def back():
                p1_write(j, b).start()

            if is_last:
                @pl.when(j < NBLK)
                def _gf():
                    front()
                plsc.subcore_barrier()

                @pl.when(j < NBLK)
                def _gb():
                    back()
            else:
                front()
                plsc.subcore_barrier()
                back()

        p1_read(sid, 0).start()

        @pl.loop(0, BLK_PER_SUB - 1, step=2)
        def _p1_loop(jj0):
            p1_step(jj0, 0, is_last=False)
            p1_step(jj0 + 1, 1, is_last=False)

        p1_step(BLK_PER_SUB - 1, 0, is_last=True)

        jlast0 = (BLK_PER_SUB - 1) * NUM_SUBCORES + sid

        @pl.when(jlast0 < NBLK)
        def _w0a():
            p1_write(jlast0, 0).wait()

        @pl.when(jlast0 >= NBLK)
        def _w0b():
            p1_write(jlast0 - 2 * NUM_SUBCORES, 0).wait()

        p1_write((BLK_PER_SUB - 2) * NUM_SUBCORES + sid, 1).wait()

        @pl.when(sid == 0)
        def _tail():
            tv = tr_v.at[0].at[pl.ds(0, TAIL // 2), :]
            pltpu.sync_copy(tail_hbm, tv)
            pltpu.sync_copy(tv, my_scr.at[pl.ds(VA // 2, TAIL // 2)])

        plsc.subcore_barrier()

        w = sid * NUM_CORES + cid
        b0 = w * BPW

        pltpu.sync_copy(lt_hbm, lt_v)
        pltpu.sync_copy(actions_hbm.at[:, pl.ds(b0, BPW)], idx_v)
        zvec = jnp.zeros((LANES,), jnp.int32)
        onevec = jnp.full((LANES,), 1, jnp.int32)

        @plsc.parallel_loop(0, T, unroll=2)
        def _prep(t):
            for q in range(BPW // LANES):
                v = idx_v[t, pl.ds(q * LANES, LANES)]
                ihalf_v[t, pl.ds(q * LANES, LANES)] = v >> 1
                offv_v[t, pl.ds(q * LANES, LANES)] = (v & 1) << 6

        plsc.subcore_barrier()

        def gather(t, b):
            return pltpu.make_async_copy(
                my_scr.at[ihalf_v.at[t]], rows_v.at[b], sem_g[b])

        def wb(t, b):
            return pltpu.make_async_copy(
                out_v.at[b], out_hbm.at[t, :, :, pl.ds(b0, BPW)], sem_w[b])

        def compute(t, b):
            rows_b = rows_v.at[b]
            out_b = out_v.at[b]

            offs = [offv_v[t, pl.ds(bb * LANES, LANES)]
                    for bb in range(BPW // LANES)]

            @plsc.parallel_loop(0, EMBED_DIM, unroll=4)
            def _e_body(e):
                es = jnp.full((LANES,), e, jnp.int32)
                lt0s = plsc.load_gather(lt_v, [zvec, es])
                lt1s = plsc.load_gather(lt_v, [onevec, es])
                for bb in range(BPW // LANES):
                    col = offs[bb] + es
                    g = plsc.load_gather(rows_b, [rvec[bb], col])
                    out_b[0, e, pl.ds(bb * LANES, LANES)] = g + lt0s
                    out_b[1, e, pl.ds(bb * LANES, LANES)] = g + lt1s

            plsc.subcore_barrier()

        def step(t, b, do_wait_wb, do_gather_ahead):
            if do_gather_ahead:
                gather(t + 1, 1 - b).start()
            gather(t, b).wait()
            if do_wait_wb:
                wb(t - 2, b).wait()
            compute(t, b)
            wb(t, b).start()

        gather(0, 0).start()
        step(0, 0, do_wait_wb=False, do_gather_ahead=True)
        step(1, 1, do_wait_wb=False, do_gather_ahead=True)

        @pl.loop(2, T - 2, step=2)
        def _steady(t0):
            step(t0, 0, do_wait_wb=True, do_gather_ahead=True)
            step(t0 + 1, 1, do_wait_wb=True, do_gather_ahead=True)

        step(T - 2, 0, do_wait_wb=True, do_gather_ahead=True)
        step(T - 1, 1, do_wait_wb=True, do_gather_ahead=False)
        wb(T - 2, 0).wait()
        wb(T - 1, 1).wait()

    return action_encode


def kernel(actions, action_embed, learned_token):
    actions_t = actions.T.astype(jnp.int32)
    table_t = action_embed.T
    lt = learned_token.reshape(NUM_TOKENS, EMBED_DIM)
    tail = action_embed[VA:].reshape(TAIL // 2, 128)
    out, _ = _make_kernel()(actions_t, table_t, lt, tail)
    return jnp.transpose(out, (3, 0, 1, 2))
